# baseline (speedup 1.0000x reference)
import jax
import jax.numpy as jnp
from jax.experimental import pallas as pl
from jax.experimental.pallas import tpu as pltpu


def _gcn_block(x_ref, w_ref, b_ref, adjl_ref, adjr_ref, out_ref, h_ref):
    @pl.when(pl.program_id(0) == 0)
    def _():
        h_ref[...] = (
            jax.lax.dot_general(
                x_ref[...],
                w_ref[...],
                (((1,), (1,)), ((), ())),
                preferred_element_type=jnp.float32,
            )
            + b_ref[...]
        )

    half = h_ref.shape[0] // 2
    bi = out_ref.shape[0]
    adjl = adjl_ref[...].reshape(bi, half)
    adjr = adjr_ref[...].reshape(bi, half)
    acc = jnp.dot(
        adjl, h_ref[:half, :], preferred_element_type=jnp.float32
    ) + jnp.dot(adjr, h_ref[half:, :], preferred_element_type=jnp.float32)
    out_ref[...] = jnp.maximum(acc, 0.0)


def kernel(x, adj, W, b):
    n, din = x.shape
    dout = W.shape[0]
    bi = 400
    half = n // 2
    adj4 = adj.reshape(n, 2, 1, half)
    out = pl.pallas_call(
        _gcn_block,
        grid=(n // bi,),
        in_specs=[
            pl.BlockSpec((n, din), lambda i: (0, 0)),
            pl.BlockSpec((dout, din), lambda i: (0, 0)),
            pl.BlockSpec((1, dout), lambda i: (0, 0)),
            pl.BlockSpec((bi, 1, 1, half), lambda i: (i, 0, 0, 0)),
            pl.BlockSpec((bi, 1, 1, half), lambda i: (i, 1, 0, 0)),
        ],
        out_specs=pl.BlockSpec((bi, dout), lambda i: (i, 0)),
        out_shape=jax.ShapeDtypeStruct((n, dout), jnp.float32),
        scratch_shapes=[pltpu.VMEM((n, dout), jnp.float32)],
    )(x, W, b.reshape(1, dout), adj4, adj4)
    return (out, adj)


# speedup vs baseline: 8.5289x; 8.5289x over previous
import jax
import jax.numpy as jnp
from jax.experimental import pallas as pl
from jax.experimental.pallas import tpu as pltpu


def _gcn_block(x_ref, w_ref, b_ref, adjt_ref, adjb_ref, outt_ref, outb_ref, h_ref):
    @pl.when(pl.program_id(0) == 0)
    def _():
        h_ref[...] = (
            jax.lax.dot_general(
                x_ref[...],
                w_ref[...],
                (((1,), (1,)), ((), ())),
                preferred_element_type=jnp.float32,
            )
            + b_ref[...]
        )

    outt_ref[...] = jnp.maximum(
        jnp.dot(adjt_ref[...], h_ref[...], preferred_element_type=jnp.float32), 0.0
    )
    outb_ref[...] = jnp.maximum(
        jnp.dot(adjb_ref[...], h_ref[...], preferred_element_type=jnp.float32), 0.0
    )


def kernel(x, adj, W, b):
    n, din = x.shape
    dout = W.shape[0]
    n2 = n // 2
    bi = 200
    nblk = n2 // bi
    outt, outb = pl.pallas_call(
        _gcn_block,
        grid=(nblk,),
        in_specs=[
            pl.BlockSpec((n, din), lambda i: (0, 0)),
            pl.BlockSpec((dout, din), lambda i: (0, 0)),
            pl.BlockSpec((1, dout), lambda i: (0, 0)),
            pl.BlockSpec((bi, n), lambda i: (i, 0)),
            pl.BlockSpec((bi, n), lambda i: (i + nblk, 0)),
        ],
        out_specs=[
            pl.BlockSpec((bi, dout), lambda i: (i, 0)),
            pl.BlockSpec((bi, dout), lambda i: (i, 0)),
        ],
        out_shape=[
            jax.ShapeDtypeStruct((n2, dout), jnp.float32),
            jax.ShapeDtypeStruct((n2, dout), jnp.float32),
        ],
        scratch_shapes=[pltpu.VMEM((n, dout), jnp.float32)],
    )(x, W, b.reshape(1, dout), adj, adj)
    return (jnp.concatenate([outt, outb], axis=0), adj)


# speedup vs baseline: 8.6626x; 1.0157x over previous
import jax
import jax.numpy as jnp
from jax.experimental import pallas as pl
from jax.experimental.pallas import tpu as pltpu


def _gcn_block(x_ref, w_ref, b_ref, adj_ref, out_ref, h_ref):
    @pl.when(pl.program_id(0) == 0)
    def _():
        h_ref[...] = (
            jax.lax.dot_general(
                x_ref[...],
                w_ref[...],
                (((1,), (1,)), ((), ())),
                preferred_element_type=jnp.float32,
            )
            + b_ref[...]
        )

    out_ref[...] = jnp.maximum(
        jnp.dot(adj_ref[...], h_ref[...], preferred_element_type=jnp.float32),
        0.0,
    )


def kernel(x, adj, W, b):
    n, din = x.shape
    dout = W.shape[0]
    bi = 400
    out = pl.pallas_call(
        _gcn_block,
        grid=(n // bi,),
        in_specs=[
            pl.BlockSpec((n, din), lambda i: (0, 0)),
            pl.BlockSpec((dout, din), lambda i: (0, 0)),
            pl.BlockSpec((1, dout), lambda i: (0, 0)),
            pl.BlockSpec((bi, n), lambda i: (i, 0)),
        ],
        out_specs=pl.BlockSpec((bi, dout), lambda i: (i, 0)),
        out_shape=jax.ShapeDtypeStruct((n, dout), jnp.float32),
        scratch_shapes=[pltpu.VMEM((n, dout), jnp.float32)],
    )(x, W, b.reshape(1, dout), adj)
    return (out, adj)


# speedup vs baseline: 8.6818x; 1.0022x over previous
import jax
import jax.numpy as jnp
from jax.experimental import pallas as pl
from jax.experimental.pallas import tpu as pltpu


def _gcn_block(x_ref, w_ref, b_ref, adj_ref, out_ref, h_ref):
    @pl.when(pl.program_id(0) == 0)
    def _():
        h_ref[...] = (
            jax.lax.dot_general(
                x_ref[...],
                w_ref[...],
                (((1,), (1,)), ((), ())),
                preferred_element_type=jnp.float32,
            )
            + b_ref[...]
        )

    out_ref[...] = jnp.maximum(
        jnp.dot(adj_ref[...], h_ref[...], preferred_element_type=jnp.float32),
        0.0,
    )


def kernel(x, adj, W, b):
    n, din = x.shape
    dout = W.shape[0]
    bi = 200
    out = pl.pallas_call(
        _gcn_block,
        grid=(n // bi,),
        in_specs=[
            pl.BlockSpec((n, din), lambda i: (0, 0)),
            pl.BlockSpec((dout, din), lambda i: (0, 0)),
            pl.BlockSpec((1, dout), lambda i: (0, 0)),
            pl.BlockSpec((bi, n), lambda i: (i, 0)),
        ],
        out_specs=pl.BlockSpec((bi, dout), lambda i: (i, 0)),
        out_shape=jax.ShapeDtypeStruct((n, dout), jnp.float32),
        scratch_shapes=[pltpu.VMEM((n, dout), jnp.float32)],
    )(x, W, b.reshape(1, dout), adj)
    return (out, adj)
